# baseline (speedup 1.0000x reference)
import jax
import jax.numpy as jnp
from jax.experimental import pallas as pl

N_CLS_ = 1000
N_CTX_ = 4
DIM_ = 768
SUF_ = 72
ROW_ = (1 + N_CTX_ + SUF_) * DIM_
B_ = 8


def _concat_body(prefix_ref, ctx_ref, suffix_ref, out_ref):
    out_ref[:, 0:DIM_] = prefix_ref[...]
    out_ref[:, DIM_:(1 + N_CTX_) * DIM_] = jnp.broadcast_to(
        ctx_ref[...], (out_ref.shape[0], N_CTX_ * DIM_)
    )
    out_ref[:, (1 + N_CTX_) * DIM_:] = suffix_ref[...]


def kernel(token_prefix, ctx, token_suffix):
    n_cls = token_prefix.shape[0]
    prefix2d = token_prefix.reshape(n_cls, DIM_)
    suffix2d = token_suffix.reshape(n_cls, SUF_ * DIM_)
    ctx2d = ctx.reshape(1, N_CTX_ * DIM_)

    out2d = pl.pallas_call(
        _concat_body,
        grid=(n_cls // B_,),
        in_specs=[
            pl.BlockSpec((B_, DIM_), lambda i: (i, 0)),
            pl.BlockSpec((1, N_CTX_ * DIM_), lambda i: (0, 0)),
            pl.BlockSpec((B_, SUF_ * DIM_), lambda i: (i, 0)),
        ],
        out_specs=pl.BlockSpec((B_, ROW_), lambda i: (i, 0)),
        out_shape=jax.ShapeDtypeStruct((n_cls, ROW_), jnp.float32),
    )(prefix2d, ctx2d, suffix2d)

    return out2d.reshape(n_cls, 1 + N_CTX_ + SUF_, DIM_)
